# baseline (speedup 1.0000x reference)
import jax
import jax.numpy as jnp
from jax import lax
from jax.experimental import pallas as pl
from jax.experimental.pallas import tpu as pltpu
from jax.experimental.pallas import tpu_sc as plsc

B = 16384
HIST = 200
D = 32
NC = 2
NS = 16
NW = NC * NS
RPW = B // NW
CHUNK = 64
NCHUNK = RPW // CHUNK
GA = 128
GB = HIST - GA
NBUF = 4
VOCAB = 1000000
NCOLT = -(-VOCAB // 128)
VOCABPAD = NCOLT * 128
TPW = -(-NCOLT // NW)
NLAST = NCOLT - NW * (TPW - 1)


def _relabel_body(in_hbm, embt_hbm, outa_hbm, outb_hbm, embdm_hbm,
                  va, vb, vt, loadsems, storesem, colsems, tstoresem):
    wid = lax.axis_index("s") * NC + lax.axis_index("c")
    row0 = wid * RPW

    colb = pl.multiple_of(GA + row0 * 0, GA)

    def load(c, p):
        base = row0 + c * CHUNK
        pltpu.async_copy(in_hbm.at[pl.ds(base, CHUNK), pl.ds(0, GA)],
                         va.at[pl.ds(p * CHUNK, CHUNK)], loadsems.at[p])
        pltpu.async_copy(in_hbm.at[pl.ds(base, CHUNK), pl.ds(colb, GA)],
                         vb.at[pl.ds(p * CHUNK, CHUNK)], loadsems.at[p])

    def wait_load(p):
        pltpu.make_async_copy(in_hbm.at[pl.ds(0, CHUNK), pl.ds(0, GA)],
                              va.at[pl.ds(p * CHUNK, CHUNK)],
                              loadsems.at[p]).wait()
        pltpu.make_async_copy(in_hbm.at[pl.ds(0, CHUNK), pl.ds(0, GA)],
                              vb.at[pl.ds(p * CHUNK, CHUNK)],
                              loadsems.at[p]).wait()

    def store(c, p):
        base = row0 + c * CHUNK
        pltpu.async_copy(va.at[pl.ds(p * CHUNK, CHUNK)],
                         outa_hbm.at[pl.ds(base, CHUNK)], storesem)
        pltpu.async_copy(vb.at[pl.ds(p * CHUNK, CHUNK)],
                         outb_hbm.at[pl.ds(base, CHUNK)], storesem)

    def wait_store():
        pltpu.make_async_copy(va.at[pl.ds(0, CHUNK)],
                              outa_hbm.at[pl.ds(0, CHUNK)], storesem).wait()
        pltpu.make_async_copy(vb.at[pl.ds(0, CHUNK)],
                              outb_hbm.at[pl.ds(0, CHUNK)], storesem).wait()

    load(0, 0)
    for c in range(NCHUNK):
        p = c % 2
        if c >= 1:
            wait_store()
        if c + 1 < NCHUNK:
            load(c + 1, 1 - p)
        wait_load(p)
        store(c, p)
    wait_store()

    def col_of(t):
        return wid + NW * t

    def t_load(t, s):
        c128 = pl.multiple_of(col_of(t) * 128, 128)
        pltpu.async_copy(embt_hbm.at[:, pl.ds(c128, 128)],
                         vt.at[pl.ds(s * D, D)], colsems.at[s])

    def t_wait_load(s):
        pltpu.make_async_copy(embt_hbm.at[:, pl.ds(0, 128)],
                              vt.at[pl.ds(s * D, D)], colsems.at[s]).wait()

    def t_store(t, s):
        col = col_of(t)
        for tr in range(4):
            pltpu.async_copy(vt.at[pl.ds(s * D + tr * 8, 8)],
                             embdm_hbm.at[pl.ds((tr * NCOLT + col) * 8, 8)],
                             tstoresem)

    def t_wait_store():
        pltpu.make_async_copy(vt.at[pl.ds(0, D)],
                              embdm_hbm.at[pl.ds(0, D)], tstoresem).wait()

    t_load(0, 0)

    def qbody(q, _):
        for s in (0, 1):
            t = 2 * q + s
            if s == 0:
                t_load(t + 1, 1)
            else:
                @pl.when(jnp.logical_or(q < (TPW - 1) // 2 - 1, wid < NLAST))
                def _():
                    t_load(t + 1, 0)

            @pl.when(q >= 1)
            def _():
                t_wait_store()

            t_wait_load(s)
            t_store(t, s)
        return 0

    lax.fori_loop(0, (TPW - 1) // 2, qbody, 0)
    t_wait_store()
    last_valid = wid < NLAST

    @pl.when(last_valid)
    def _():
        t_wait_load(0)
        t_store(TPW - 1, 0)

    t_wait_store()

    @pl.when(last_valid)
    def _():
        t_wait_store()


def _tcast_body(embdm_hbm, embbf_hbm, vt, tout, csems, ssem):
    wid = lax.axis_index("s") * NC + lax.axis_index("c")
    lane = lax.iota(jnp.int32, 16)

    def col_of(t):
        return wid + NW * t

    def t_load(t, s):
        col = col_of(t)
        for tr in range(4):
            pltpu.async_copy(embdm_hbm.at[pl.ds((tr * NCOLT + col) * 8, 8)],
                             vt.at[pl.ds(s * D + tr * 8, 8)], csems.at[s])

    def t_wait_load(s):
        pltpu.make_async_copy(embdm_hbm.at[pl.ds(0, D)],
                              vt.at[pl.ds(s * D, D)], csems.at[s]).wait()

    def t_store(t, s):
        pltpu.async_copy(tout.at[pl.ds(s * 128 * D, 128 * D)],
                         embbf_hbm.at[pl.ds(col_of(t) * 128 * D, 128 * D)],
                         ssem)

    def t_wait_store():
        pltpu.make_async_copy(tout.at[pl.ds(0, 128 * D)],
                              embbf_hbm.at[pl.ds(0, 128 * D)], ssem).wait()

    def transpose_col(s):
        rows_ev = jnp.full((16,), s * D, jnp.int32) + lane * 2
        rows_od = rows_ev + 1

        def jbody(jj, _):
            for u in range(4):
                j = jj * 4 + u
                colv = jnp.full((16,), j, jnp.int32)
                ev = plsc.load_gather(vt, [rows_ev, colv])
                od = plsc.load_gather(vt, [rows_od, colv])
                ab = plsc.pack(ev, od, format=plsc.PackFormat.INTERLEAVED)
                tout[pl.ds(s * 128 * D + j * D, D)] = ab
            return 0

        lax.fori_loop(0, 128 // 4, jbody, 0)

    t_load(0, 0)

    def qbody(q, _):
        for s in (0, 1):
            t = 2 * q + s
            if s == 0:
                t_load(t + 1, 1)
            else:
                @pl.when(jnp.logical_or(q < (TPW - 1) // 2 - 1, wid < NLAST))
                def _():
                    t_load(t + 1, 0)

            @pl.when(q >= 1)
            def _():
                t_wait_store()

            t_wait_load(s)
            transpose_col(s)
            t_store(t, s)
        return 0

    lax.fori_loop(0, (TPW - 1) // 2, qbody, 0)
    t_wait_store()
    last_valid = wid < NLAST

    @pl.when(last_valid)
    def _():
        t_wait_load(0)
        transpose_col(0)
        t_store(TPW - 1, 0)

    t_wait_store()

    @pl.when(last_valid)
    def _():
        t_wait_store()


def _gather_body(idxa_hbm, idxb_hbm, emb_hbm, out_hbm,
                 idxa_v, idxb_v, rows_a, rows_b, emb0_v, out_v, sems):
    wid = lax.axis_index("s") * NC + lax.axis_index("c")
    row0 = wid * RPW
    pltpu.sync_copy(emb_hbm.at[0], emb0_v)
    e0_ev, e0_od = plsc.unpack(emb0_v[...], format=plsc.PackFormat.INTERLEAVED)
    lane = lax.iota(jnp.int32, 16)
    tail_mask = lane >= 8
    sc_even = lane * 2
    sc_odd = lane * 2 + 1
    zero_v = jnp.zeros((16,), jnp.float32)
    hist_v = jnp.full((16,), float(HIST), jnp.float32)

    def issue(r, s):
        pltpu.async_copy(emb_hbm.at[idxa_v.at[r]],
                         rows_a.at[pl.ds(s * GA, GA)], sems.at[s])
        pltpu.async_copy(emb_hbm.at[idxb_v.at[r, pl.ds(0, GB)]],
                         rows_b.at[pl.ds(s * GB, GB)], sems.at[s])

    def drain(s):
        pltpu.make_async_copy(emb_hbm.at[pl.ds(0, GA)],
                              rows_a.at[pl.ds(s * GA, GA)], sems.at[s]).wait()
        pltpu.make_async_copy(emb_hbm.at[pl.ds(0, GB)],
                              rows_b.at[pl.ds(s * GB, GB)], sems.at[s]).wait()

    def chunk_body(c, _):
        base = row0 + c * CHUNK
        pltpu.sync_copy(idxa_hbm.at[pl.ds(base, CHUNK)], idxa_v)
        pltpu.sync_copy(idxb_hbm.at[pl.ds(base, CHUNK)], idxb_v)
        for s in range(NBUF - 1):
            issue(s, s)

        def block_body(q, _):
            r0 = q * NBUF
            for s in range(NBUF):
                r = r0 + s
                nxt = r + (NBUF - 1)

                @pl.when(nxt < CHUNK)
                def _():
                    issue(nxt, (s + NBUF - 1) % NBUF)

                cnt = jnp.zeros((16,), jnp.int32)
                for k in range(8):
                    iszero = idxa_v[r, pl.ds(k * 16, 16)] == 0
                    cnt = cnt + plsc.all_reduce_population_count(iszero)
                for k in range(4):
                    iszero = idxb_v[r, pl.ds(k * 16, 16)] == 0
                    cnt = cnt + plsc.all_reduce_population_count(iszero)
                tail_zero = jnp.logical_and(idxb_v[r, pl.ds(GB - 16, 16)] == 0,
                                            tail_mask)
                cnt = cnt + plsc.all_reduce_population_count(tail_zero)
                nzv = cnt.astype(jnp.float32)

                drain(s)

                def acc_a(t, carry):
                    ev, od = carry
                    j0 = s * GA + t * 8
                    for u in range(8):
                        a, b = plsc.unpack(rows_a[j0 + u, :],
                                           format=plsc.PackFormat.INTERLEAVED)
                        ev = ev + a
                        od = od + b
                    return ev, od

                ev, od = lax.fori_loop(0, GA // 8, acc_a, (zero_v, zero_v))

                def acc_b(t, carry):
                    ev, od = carry
                    j0 = s * GB + t * 8
                    for u in range(8):
                        a, b = plsc.unpack(rows_b[j0 + u, :],
                                           format=plsc.PackFormat.INTERLEAVED)
                        ev = ev + a
                        od = od + b
                    return ev, od

                ev, od = lax.fori_loop(0, GB // 8, acc_b, (ev, od))
                denom = (hist_v - nzv) + 1e-8
                rowv = jnp.full((16,), r, jnp.int32)
                plsc.store_scatter(out_v, [rowv, sc_even],
                                   (ev - nzv * e0_ev) / denom)
                plsc.store_scatter(out_v, [rowv, sc_odd],
                                   (od - nzv * e0_od) / denom)
            return 0

        lax.fori_loop(0, CHUNK // NBUF, block_body, 0)
        pltpu.sync_copy(out_v, out_hbm.at[pl.ds(base, CHUNK)])
        return 0

    lax.fori_loop(0, NCHUNK, chunk_body, 0)


def kernel(inputs, embeddings):
    mesh = plsc.VectorSubcoreMesh(core_axis_name="c", subcore_axis_name="s",
                                  num_cores=NC, num_subcores=NS)
    relabel = pl.kernel(
        _relabel_body,
        out_type=(jax.ShapeDtypeStruct((B, GA), jnp.int32),
                  jax.ShapeDtypeStruct((B, GA), jnp.int32),
                  jax.ShapeDtypeStruct((4 * NCOLT * 8, 128), jnp.float32)),
        mesh=mesh,
        compiler_params=pltpu.CompilerParams(needs_layout_passes=False,
                                             use_tc_tiling_on_sc=True),
        scratch_types=[
            pltpu.VMEM((2 * CHUNK, GA), jnp.int32),
            pltpu.VMEM((2 * CHUNK, GA), jnp.int32),
            pltpu.VMEM((2 * D, 128), jnp.float32),
            pltpu.SemaphoreType.DMA((2,)),
            pltpu.SemaphoreType.DMA,
            pltpu.SemaphoreType.DMA((2,)),
            pltpu.SemaphoreType.DMA,
        ],
    )
    tcast = pl.kernel(
        _tcast_body,
        out_type=jax.ShapeDtypeStruct((VOCABPAD * D,), jnp.bfloat16),
        mesh=mesh,
        compiler_params=pltpu.CompilerParams(needs_layout_passes=False,
                                             use_tc_tiling_on_sc=False),
        scratch_types=[
            pltpu.VMEM((2 * D, 128), jnp.float32),
            pltpu.VMEM((2 * 128 * D,), jnp.bfloat16),
            pltpu.SemaphoreType.DMA((2,)),
            pltpu.SemaphoreType.DMA,
        ],
    )
    gather = pl.kernel(
        _gather_body,
        out_type=jax.ShapeDtypeStruct((B, D), jnp.float32),
        mesh=mesh,
        compiler_params=pltpu.CompilerParams(needs_layout_passes=False,
                                             use_tc_tiling_on_sc=False),
        scratch_types=[
            pltpu.VMEM((CHUNK, GA), jnp.int32),
            pltpu.VMEM((CHUNK, GA), jnp.int32),
            pltpu.VMEM((NBUF * GA, D), jnp.bfloat16),
            pltpu.VMEM((NBUF * GB, D), jnp.bfloat16),
            pltpu.VMEM((D,), jnp.bfloat16),
            pltpu.VMEM((CHUNK, D), jnp.float32),
            pltpu.SemaphoreType.DMA((NBUF,)),
        ],
    )
    idxa, idxb, embdm = relabel(inputs, embeddings.T)
    embbf = tcast(embdm)
    return gather(idxa, idxb, embbf.reshape(VOCABPAD, D))


# speedup vs baseline: 1.8276x; 1.8276x over previous
import jax
import jax.numpy as jnp
from jax import lax
from jax.experimental import pallas as pl
from jax.experimental.pallas import tpu as pltpu
from jax.experimental.pallas import tpu_sc as plsc

B = 16384
HIST = 200
D = 32
NC = 2
NS = 16
NW = NC * NS
RPW = B // NW
CHUNK = 64
NCHUNK = RPW // CHUNK
GA = 128
GB = HIST - GA
NBUF = 4


def _detile_body(in_hbm, outa_hbm, outb_hbm, va, vb, loadsems, storesem):
    wid = lax.axis_index("s") * NC + lax.axis_index("c")
    row0 = wid * RPW

    colb = pl.multiple_of(GA + row0 * 0, GA)

    def load(c, p):
        base = row0 + c * CHUNK
        pltpu.async_copy(in_hbm.at[pl.ds(base, CHUNK), pl.ds(0, GA)],
                         va.at[pl.ds(p * CHUNK, CHUNK)], loadsems.at[p])
        pltpu.async_copy(in_hbm.at[pl.ds(base, CHUNK), pl.ds(colb, GA)],
                         vb.at[pl.ds(p * CHUNK, CHUNK)], loadsems.at[p])

    def wait_load(p):
        pltpu.make_async_copy(in_hbm.at[pl.ds(0, CHUNK), pl.ds(0, GA)],
                              va.at[pl.ds(p * CHUNK, CHUNK)],
                              loadsems.at[p]).wait()
        pltpu.make_async_copy(in_hbm.at[pl.ds(0, CHUNK), pl.ds(0, GA)],
                              vb.at[pl.ds(p * CHUNK, CHUNK)],
                              loadsems.at[p]).wait()

    def store(c, p):
        base = row0 + c * CHUNK
        pltpu.async_copy(va.at[pl.ds(p * CHUNK, CHUNK)],
                         outa_hbm.at[pl.ds(base, CHUNK)], storesem)
        pltpu.async_copy(vb.at[pl.ds(p * CHUNK, CHUNK)],
                         outb_hbm.at[pl.ds(base, CHUNK)], storesem)

    def wait_store():
        pltpu.make_async_copy(va.at[pl.ds(0, CHUNK)],
                              outa_hbm.at[pl.ds(0, CHUNK)], storesem).wait()
        pltpu.make_async_copy(vb.at[pl.ds(0, CHUNK)],
                              outb_hbm.at[pl.ds(0, CHUNK)], storesem).wait()

    load(0, 0)
    for c in range(NCHUNK):
        p = c % 2
        if c >= 1:
            wait_store()
        if c + 1 < NCHUNK:
            load(c + 1, 1 - p)
        wait_load(p)
        store(c, p)
    wait_store()


def _gather_body(idxa_hbm, idxb_hbm, emb_hbm, out_hbm,
                 idxa_v, idxb_v, rows_a, rows_b, emb0_v, out_v, sems):
    wid = lax.axis_index("s") * NC + lax.axis_index("c")
    row0 = wid * RPW
    pltpu.sync_copy(emb_hbm.at[0], emb0_v)
    e0_lo = emb0_v[pl.ds(0, 16)]
    e0_hi = emb0_v[pl.ds(16, 16)]
    lane = lax.iota(jnp.int32, 16)
    tail_mask = lane >= 8
    zero_v = jnp.zeros((16,), jnp.float32)
    hist_v = jnp.full((16,), float(HIST), jnp.float32)

    def issue(r, s):
        pltpu.async_copy(emb_hbm.at[idxa_v.at[r]],
                         rows_a.at[pl.ds(s * GA, GA)], sems.at[s])
        pltpu.async_copy(emb_hbm.at[idxb_v.at[r, pl.ds(0, GB)]],
                         rows_b.at[pl.ds(s * GB, GB)], sems.at[s])

    def drain(s):
        pltpu.make_async_copy(emb_hbm.at[pl.ds(0, GA)],
                              rows_a.at[pl.ds(s * GA, GA)], sems.at[s]).wait()
        pltpu.make_async_copy(emb_hbm.at[pl.ds(0, GB)],
                              rows_b.at[pl.ds(s * GB, GB)], sems.at[s]).wait()

    def chunk_body(c, _):
        base = row0 + c * CHUNK
        pltpu.sync_copy(idxa_hbm.at[pl.ds(base, CHUNK)], idxa_v)
        pltpu.sync_copy(idxb_hbm.at[pl.ds(base, CHUNK)], idxb_v)
        for s in range(NBUF - 1):
            issue(s, s)

        def block_body(q, _):
            r0 = q * NBUF
            for s in range(NBUF):
                r = r0 + s
                nxt = r + (NBUF - 1)

                @pl.when(nxt < CHUNK)
                def _():
                    issue(nxt, (s + NBUF - 1) % NBUF)

                cnt = jnp.zeros((16,), jnp.int32)
                for k in range(8):
                    iszero = idxa_v[r, pl.ds(k * 16, 16)] == 0
                    cnt = cnt + plsc.all_reduce_population_count(iszero)
                for k in range(4):
                    iszero = idxb_v[r, pl.ds(k * 16, 16)] == 0
                    cnt = cnt + plsc.all_reduce_population_count(iszero)
                tail_zero = jnp.logical_and(idxb_v[r, pl.ds(GB - 16, 16)] == 0,
                                            tail_mask)
                cnt = cnt + plsc.all_reduce_population_count(tail_zero)
                nzv = cnt.astype(jnp.float32)

                drain(s)

                def acc_a(t, carry):
                    lo, hi = carry
                    j0 = s * GA + t * 8
                    for u in range(8):
                        lo = lo + rows_a[j0 + u, pl.ds(0, 16)]
                        hi = hi + rows_a[j0 + u, pl.ds(16, 16)]
                    return lo, hi

                lo, hi = lax.fori_loop(0, GA // 8, acc_a, (zero_v, zero_v))

                def acc_b(t, carry):
                    lo, hi = carry
                    j0 = s * GB + t * 8
                    for u in range(8):
                        lo = lo + rows_b[j0 + u, pl.ds(0, 16)]
                        hi = hi + rows_b[j0 + u, pl.ds(16, 16)]
                    return lo, hi

                lo, hi = lax.fori_loop(0, GB // 8, acc_b, (lo, hi))
                denom = (hist_v - nzv) + 1e-8
                out_v[r, pl.ds(0, 16)] = (lo - nzv * e0_lo) / denom
                out_v[r, pl.ds(16, 16)] = (hi - nzv * e0_hi) / denom
            return 0

        lax.fori_loop(0, CHUNK // NBUF, block_body, 0)
        pltpu.sync_copy(out_v, out_hbm.at[pl.ds(base, CHUNK)])
        return 0

    lax.fori_loop(0, NCHUNK, chunk_body, 0)


def kernel(inputs, embeddings):
    mesh = plsc.VectorSubcoreMesh(core_axis_name="c", subcore_axis_name="s",
                                  num_cores=NC, num_subcores=NS)
    detile = pl.kernel(
        _detile_body,
        out_type=(jax.ShapeDtypeStruct((B, GA), jnp.int32),
                  jax.ShapeDtypeStruct((B, GA), jnp.int32)),
        mesh=mesh,
        compiler_params=pltpu.CompilerParams(needs_layout_passes=False,
                                             use_tc_tiling_on_sc=True),
        scratch_types=[
            pltpu.VMEM((2 * CHUNK, GA), jnp.int32),
            pltpu.VMEM((2 * CHUNK, GA), jnp.int32),
            pltpu.SemaphoreType.DMA((2,)),
            pltpu.SemaphoreType.DMA,
        ],
    )
    gather = pl.kernel(
        _gather_body,
        out_type=jax.ShapeDtypeStruct((B, D), jnp.float32),
        mesh=mesh,
        compiler_params=pltpu.CompilerParams(needs_layout_passes=False,
                                             use_tc_tiling_on_sc=False),
        scratch_types=[
            pltpu.VMEM((CHUNK, GA), jnp.int32),
            pltpu.VMEM((CHUNK, GA), jnp.int32),
            pltpu.VMEM((NBUF * GA, D), jnp.float32),
            pltpu.VMEM((NBUF * GB, D), jnp.float32),
            pltpu.VMEM((D,), jnp.float32),
            pltpu.VMEM((CHUNK, D), jnp.float32),
            pltpu.SemaphoreType.DMA((NBUF,)),
        ],
    )
    idxa, idxb = detile(inputs)
    return gather(idxa, idxb, embeddings)
